# Initial kernel scaffold; baseline (speedup 1.0000x reference)
import jax
import jax.numpy as jnp
from jax.experimental import pallas as pl


def kernel(x, edge_index, W1, b1, W2, b2):
    raise NotImplementedError("write your pallas kernel here")


# speedup vs baseline: 8.5918x; 8.5918x over previous
import functools

import jax
import jax.numpy as jnp
from jax import lax
from jax.experimental import pallas as pl
from jax.experimental.pallas import tpu as pltpu
from jax.experimental.pallas import tpu_sc as plsc

NC = 2
NS = 16
NW = NC * NS
LANES = 16
EB = 64
ZR = 8

_mesh = plsc.VectorSubcoreMesh(core_axis_name="c", subcore_axis_name="s")


def _sc_degree(dst2d, npad, cpw):
    rpt = npad // NS

    @functools.partial(
        pl.kernel,
        mesh=_mesh,
        out_type=jax.ShapeDtypeStruct((NC * npad, LANES), jnp.float32),
        scratch_types=[
            pltpu.VMEM((cpw, EB), jnp.int32),
            pltpu.VMEM((EB, LANES), jnp.float32),
            pltpu.VMEM((ZR, LANES), jnp.float32),
            pltpu.VMEM_SHARED((npad, LANES), jnp.float32),
        ],
    )
    def kdeg(dst_hbm, out_hbm, dst_v, ones_v, z_v, deg_sh):
        c = lax.axis_index("c")
        s = lax.axis_index("s")
        w = s * NC + c
        base = s * rpt
        pltpu.sync_copy(dst_hbm.at[pl.ds(w * cpw, cpw)], dst_v)

        @pl.loop(0, EB)
        def _fill_ones(i):
            ones_v[i, :] = jnp.full((LANES,), 1.0, jnp.float32)

        @pl.loop(0, ZR)
        def _fill_zero(i):
            z_v[i, :] = jnp.zeros((LANES,), jnp.float32)

        @pl.loop(0, rpt, step=ZR)
        def _zero_acc(r):
            pltpu.sync_copy(z_v, deg_sh.at[pl.ds(base + r, ZR)])

        plsc.subcore_barrier()

        @pl.loop(0, cpw)
        def _edges(t):
            pltpu.sync_copy(ones_v, deg_sh.at[dst_v.at[t]], add=True)

        plsc.subcore_barrier()
        pltpu.sync_copy(deg_sh.at[pl.ds(base, rpt)],
                        out_hbm.at[pl.ds(c * npad + base, rpt)])

    return kdeg(dst2d)


IG = 32


def _sc_aggregate(y, src2d, dst2d, npad, cpw):
    rpt = npad // NS

    @functools.partial(
        pl.kernel,
        mesh=_mesh,
        out_type=jax.ShapeDtypeStruct((NC * npad, 128), jnp.float32),
        scratch_types=[
            pltpu.VMEM((IG, EB), jnp.int32),
            pltpu.VMEM((IG, EB), jnp.int32),
            pltpu.VMEM((EB, 128), jnp.float32),
            pltpu.VMEM((EB, 128), jnp.float32),
            pltpu.VMEM((ZR, 128), jnp.float32),
            pltpu.VMEM_SHARED((npad, 128), jnp.float32),
            pltpu.SemaphoreType.DMA,
            pltpu.SemaphoreType.DMA,
        ],
    )
    def kagg(y_hbm, src_hbm, dst_hbm, out_hbm,
             src_v, dst_v, buf0, buf1, z_v, acc_sh, sem0, sem1):
        c = lax.axis_index("c")
        s = lax.axis_index("s")
        w = s * NC + c
        base = s * rpt

        @pl.loop(0, ZR)
        def _fill_zero(i):
            @pl.loop(0, 128, step=LANES)
            def _fz(j):
                z_v[i, pl.ds(j, LANES)] = jnp.zeros((LANES,), jnp.float32)

        @pl.loop(0, rpt, step=ZR)
        def _zero_acc(r):
            pltpu.sync_copy(z_v, acc_sh.at[pl.ds(base + r, ZR)])

        plsc.subcore_barrier()

        @pl.loop(0, cpw, step=IG)
        def _groups(g):
            pltpu.sync_copy(src_hbm.at[pl.ds(w * cpw + g, IG)], src_v)
            pltpu.sync_copy(dst_hbm.at[pl.ds(w * cpw + g, IG)], dst_v)

            pltpu.make_async_copy(y_hbm.at[src_v.at[0]], buf0, sem0).start()

            @pl.loop(0, IG, step=2)
            def _edges(t):
                pltpu.make_async_copy(y_hbm.at[src_v.at[t]], buf0, sem0).wait()
                pltpu.make_async_copy(
                    y_hbm.at[src_v.at[t + 1]], buf1, sem1).start()
                pltpu.sync_copy(buf0, acc_sh.at[dst_v.at[t]], add=True)
                pltpu.make_async_copy(y_hbm.at[src_v.at[t + 1]], buf1, sem1).wait()

                @pl.when(t + 2 < IG)
                def _next():
                    pltpu.make_async_copy(
                        y_hbm.at[src_v.at[t + 2]], buf0, sem0).start()

                pltpu.sync_copy(buf1, acc_sh.at[dst_v.at[t + 1]], add=True)

        plsc.subcore_barrier()
        pltpu.sync_copy(acc_sh.at[pl.ds(base, rpt)],
                        out_hbm.at[pl.ds(c * npad + base, rpt)])

    return kagg(y, src2d, dst2d)


def _tc_matmul(xp, w, npad):
    bn = 1024

    def body(x_ref, w_ref, o_ref):
        o_ref[...] = lax.dot_general(
            x_ref[...], w_ref[...], (((1,), (0,)), ((), ())),
            precision=lax.Precision.HIGHEST,
            preferred_element_type=jnp.float32)

    return pl.pallas_call(
        body,
        grid=(npad // bn,),
        in_specs=[pl.BlockSpec((bn, 128), lambda i: (i, 0)),
                  pl.BlockSpec((128, 128), lambda i: (0, 0))],
        out_specs=pl.BlockSpec((bn, 128), lambda i: (i, 0)),
        out_shape=jax.ShapeDtypeStruct((npad, 128), jnp.float32),
    )(xp, w)


def _tc_scale(deg2, xw, npad):
    bn = 1024
    nb = npad // bn

    def body(d0_ref, d1_ref, xw_ref, y_ref, di_ref):
        deg = d0_ref[...][:, 0] + d1_ref[...][:, 0] + 1.0
        dinv = lax.rsqrt(deg)
        di_ref[...] = dinv
        y_ref[...] = xw_ref[...] * dinv[:, None]

    return pl.pallas_call(
        body,
        grid=(nb,),
        in_specs=[pl.BlockSpec((bn, LANES), lambda i: (i, 0)),
                  pl.BlockSpec((bn, LANES), lambda i: (i + nb, 0)),
                  pl.BlockSpec((bn, 128), lambda i: (i, 0))],
        out_specs=[pl.BlockSpec((bn, 128), lambda i: (i, 0)),
                   pl.BlockSpec((bn,), lambda i: (i,))],
        out_shape=[jax.ShapeDtypeStruct((npad, 128), jnp.float32),
                   jax.ShapeDtypeStruct((npad,), jnp.float32)],
    )(deg2, deg2, xw)


def _tc_mid(acc2, y1, dinv, b1, w2, npad):
    bn = 1024
    nb = npad // bn

    def body(a0_ref, a1_ref, y_ref, di_ref, b_ref, w_ref, o_ref):
        di = di_ref[...]
        h = (a0_ref[...] + a1_ref[...] + y_ref[...]) * di[:, None] \
            + b_ref[...][None, :]
        hw = lax.dot_general(
            h, w_ref[...], (((1,), (0,)), ((), ())),
            precision=lax.Precision.HIGHEST,
            preferred_element_type=jnp.float32)
        o_ref[...] = hw * di[:, None]

    return pl.pallas_call(
        body,
        grid=(nb,),
        in_specs=[pl.BlockSpec((bn, 128), lambda i: (i, 0)),
                  pl.BlockSpec((bn, 128), lambda i: (i + nb, 0)),
                  pl.BlockSpec((bn, 128), lambda i: (i, 0)),
                  pl.BlockSpec((bn,), lambda i: (i,)),
                  pl.BlockSpec((128,), lambda i: (0,)),
                  pl.BlockSpec((128, 128), lambda i: (0, 0))],
        out_specs=pl.BlockSpec((bn, 128), lambda i: (i, 0)),
        out_shape=jax.ShapeDtypeStruct((npad, 128), jnp.float32),
    )(acc2, acc2, y1, dinv, b1, w2)


def _tc_final(acc2, y2, dinv, b2, npad):
    bn = 1024
    nb = npad // bn

    def body(a0_ref, a1_ref, y_ref, di_ref, b_ref, o_ref):
        o_ref[...] = (a0_ref[...] + a1_ref[...] + y_ref[...]) \
            * di_ref[...][:, None] + b_ref[...][None, :]

    return pl.pallas_call(
        body,
        grid=(nb,),
        in_specs=[pl.BlockSpec((bn, 128), lambda i: (i, 0)),
                  pl.BlockSpec((bn, 128), lambda i: (i + nb, 0)),
                  pl.BlockSpec((bn, 128), lambda i: (i, 0)),
                  pl.BlockSpec((bn,), lambda i: (i,)),
                  pl.BlockSpec((128,), lambda i: (0,))],
        out_specs=pl.BlockSpec((bn, 128), lambda i: (i, 0)),
        out_shape=jax.ShapeDtypeStruct((npad, 128), jnp.float32),
    )(acc2, acc2, y2, dinv, b2)


def kernel(x, edge_index, W1, b1, W2, b2):
    n, d = x.shape
    e = edge_index.shape[1]
    npad = -(-(n + 1) // 2048) * 2048
    cpw = -(-e // (NW * EB))
    cpw = -(-cpw // IG) * IG
    ep = NW * cpw * EB

    src = edge_index[0]
    dst = edge_index[1]
    pad = ep - e
    src2d = jnp.concatenate(
        [src, jnp.zeros((pad,), jnp.int32)]).reshape(NW * cpw, EB)
    dst2d = jnp.concatenate(
        [dst, jnp.full((pad,), n, jnp.int32)]).reshape(NW * cpw, EB)
    xp = jnp.pad(x, ((0, npad - n), (0, 0)))

    deg2 = _sc_degree(dst2d, npad, cpw)
    xw1 = _tc_matmul(xp, W1, npad)
    y1, dinv = _tc_scale(deg2, xw1, npad)
    acc1 = _sc_aggregate(y1, src2d, dst2d, npad, cpw)
    y2 = _tc_mid(acc1, y1, dinv, b1, W2, npad)
    acc2 = _sc_aggregate(y2, src2d, dst2d, npad, cpw)
    outp = _tc_final(acc2, y2, dinv, b2, npad)
    return outp[:n]


# speedup vs baseline: 9.3586x; 1.0893x over previous
import functools

import jax
import jax.numpy as jnp
from jax import lax
from jax.experimental import pallas as pl
from jax.experimental.pallas import tpu as pltpu
from jax.experimental.pallas import tpu_sc as plsc

NC = 2
NS = 16
NW = NC * NS
LANES = 16
EB = 128
ZR = 8

_mesh = plsc.VectorSubcoreMesh(core_axis_name="c", subcore_axis_name="s")


def _sc_degree(dst2d, npad, cpw):
    rpt = npad // NS

    @functools.partial(
        pl.kernel,
        mesh=_mesh,
        out_type=jax.ShapeDtypeStruct((NC * npad, LANES), jnp.float32),
        scratch_types=[
            pltpu.VMEM((cpw, EB), jnp.int32),
            pltpu.VMEM((EB, LANES), jnp.float32),
            pltpu.VMEM((ZR, LANES), jnp.float32),
            pltpu.VMEM_SHARED((npad, LANES), jnp.float32),
        ],
    )
    def kdeg(dst_hbm, out_hbm, dst_v, ones_v, z_v, deg_sh):
        c = lax.axis_index("c")
        s = lax.axis_index("s")
        w = s * NC + c
        base = s * rpt
        pltpu.sync_copy(dst_hbm.at[pl.ds(w * cpw, cpw)], dst_v)

        @pl.loop(0, EB)
        def _fill_ones(i):
            ones_v[i, :] = jnp.full((LANES,), 1.0, jnp.float32)

        @pl.loop(0, ZR)
        def _fill_zero(i):
            z_v[i, :] = jnp.zeros((LANES,), jnp.float32)

        @pl.loop(0, rpt, step=ZR)
        def _zero_acc(r):
            pltpu.sync_copy(z_v, deg_sh.at[pl.ds(base + r, ZR)])

        plsc.subcore_barrier()

        @pl.loop(0, cpw)
        def _edges(t):
            pltpu.sync_copy(ones_v, deg_sh.at[dst_v.at[t]], add=True)

        plsc.subcore_barrier()
        pltpu.sync_copy(deg_sh.at[pl.ds(base, rpt)],
                        out_hbm.at[pl.ds(c * npad + base, rpt)])

    return kdeg(dst2d)


IG = 16


def _sc_aggregate(y, src2d, dst2d, npad, cpw):
    rpt = npad // NS

    @functools.partial(
        pl.kernel,
        mesh=_mesh,
        out_type=jax.ShapeDtypeStruct((NC * npad, 128), jnp.float32),
        scratch_types=[
            pltpu.VMEM((IG, EB), jnp.int32),
            pltpu.VMEM((IG, EB), jnp.int32),
            pltpu.VMEM((EB, 128), jnp.float32),
            pltpu.VMEM((EB, 128), jnp.float32),
            pltpu.VMEM((ZR, 128), jnp.float32),
            pltpu.VMEM_SHARED((npad, 128), jnp.float32),
            pltpu.SemaphoreType.DMA,
            pltpu.SemaphoreType.DMA,
        ],
    )
    def kagg(y_hbm, src_hbm, dst_hbm, out_hbm,
             src_v, dst_v, buf0, buf1, z_v, acc_sh, sem0, sem1):
        c = lax.axis_index("c")
        s = lax.axis_index("s")
        w = s * NC + c
        base = s * rpt

        @pl.loop(0, ZR)
        def _fill_zero(i):
            @pl.loop(0, 128, step=LANES)
            def _fz(j):
                z_v[i, pl.ds(j, LANES)] = jnp.zeros((LANES,), jnp.float32)

        @pl.loop(0, rpt, step=ZR)
        def _zero_acc(r):
            pltpu.sync_copy(z_v, acc_sh.at[pl.ds(base + r, ZR)])

        plsc.subcore_barrier()

        @pl.loop(0, cpw, step=IG)
        def _groups(g):
            pltpu.sync_copy(src_hbm.at[pl.ds(w * cpw + g, IG)], src_v)
            pltpu.sync_copy(dst_hbm.at[pl.ds(w * cpw + g, IG)], dst_v)

            pltpu.make_async_copy(y_hbm.at[src_v.at[0]], buf0, sem0).start()

            @pl.loop(0, IG, step=2)
            def _edges(t):
                pltpu.make_async_copy(y_hbm.at[src_v.at[t]], buf0, sem0).wait()
                pltpu.make_async_copy(
                    y_hbm.at[src_v.at[t + 1]], buf1, sem1).start()
                pltpu.sync_copy(buf0, acc_sh.at[dst_v.at[t]], add=True)
                pltpu.make_async_copy(y_hbm.at[src_v.at[t + 1]], buf1, sem1).wait()

                @pl.when(t + 2 < IG)
                def _next():
                    pltpu.make_async_copy(
                        y_hbm.at[src_v.at[t + 2]], buf0, sem0).start()

                pltpu.sync_copy(buf1, acc_sh.at[dst_v.at[t + 1]], add=True)

        plsc.subcore_barrier()
        pltpu.sync_copy(acc_sh.at[pl.ds(base, rpt)],
                        out_hbm.at[pl.ds(c * npad + base, rpt)])

    return kagg(y, src2d, dst2d)


def _tc_matmul(xp, w, npad):
    bn = 1024

    def body(x_ref, w_ref, o_ref):
        o_ref[...] = lax.dot_general(
            x_ref[...], w_ref[...], (((1,), (0,)), ((), ())),
            precision=lax.Precision.HIGHEST,
            preferred_element_type=jnp.float32)

    return pl.pallas_call(
        body,
        grid=(npad // bn,),
        in_specs=[pl.BlockSpec((bn, 128), lambda i: (i, 0)),
                  pl.BlockSpec((128, 128), lambda i: (0, 0))],
        out_specs=pl.BlockSpec((bn, 128), lambda i: (i, 0)),
        out_shape=jax.ShapeDtypeStruct((npad, 128), jnp.float32),
    )(xp, w)


def _tc_scale(deg2, xw, npad):
    bn = 1024
    nb = npad // bn

    def body(d0_ref, d1_ref, xw_ref, y_ref, di_ref):
        deg = d0_ref[...][:, 0] + d1_ref[...][:, 0] + 1.0
        dinv = lax.rsqrt(deg)
        di_ref[...] = dinv
        y_ref[...] = xw_ref[...] * dinv[:, None]

    return pl.pallas_call(
        body,
        grid=(nb,),
        in_specs=[pl.BlockSpec((bn, LANES), lambda i: (i, 0)),
                  pl.BlockSpec((bn, LANES), lambda i: (i + nb, 0)),
                  pl.BlockSpec((bn, 128), lambda i: (i, 0))],
        out_specs=[pl.BlockSpec((bn, 128), lambda i: (i, 0)),
                   pl.BlockSpec((bn,), lambda i: (i,))],
        out_shape=[jax.ShapeDtypeStruct((npad, 128), jnp.float32),
                   jax.ShapeDtypeStruct((npad,), jnp.float32)],
    )(deg2, deg2, xw)


def _tc_mid(acc2, y1, dinv, b1, w2, npad):
    bn = 1024
    nb = npad // bn

    def body(a0_ref, a1_ref, y_ref, di_ref, b_ref, w_ref, o_ref):
        di = di_ref[...]
        h = (a0_ref[...] + a1_ref[...] + y_ref[...]) * di[:, None] \
            + b_ref[...][None, :]
        hw = lax.dot_general(
            h, w_ref[...], (((1,), (0,)), ((), ())),
            precision=lax.Precision.HIGHEST,
            preferred_element_type=jnp.float32)
        o_ref[...] = hw * di[:, None]

    return pl.pallas_call(
        body,
        grid=(nb,),
        in_specs=[pl.BlockSpec((bn, 128), lambda i: (i, 0)),
                  pl.BlockSpec((bn, 128), lambda i: (i + nb, 0)),
                  pl.BlockSpec((bn, 128), lambda i: (i, 0)),
                  pl.BlockSpec((bn,), lambda i: (i,)),
                  pl.BlockSpec((128,), lambda i: (0,)),
                  pl.BlockSpec((128, 128), lambda i: (0, 0))],
        out_specs=pl.BlockSpec((bn, 128), lambda i: (i, 0)),
        out_shape=jax.ShapeDtypeStruct((npad, 128), jnp.float32),
    )(acc2, acc2, y1, dinv, b1, w2)


def _tc_final(acc2, y2, dinv, b2, npad):
    bn = 1024
    nb = npad // bn

    def body(a0_ref, a1_ref, y_ref, di_ref, b_ref, o_ref):
        o_ref[...] = (a0_ref[...] + a1_ref[...] + y_ref[...]) \
            * di_ref[...][:, None] + b_ref[...][None, :]

    return pl.pallas_call(
        body,
        grid=(nb,),
        in_specs=[pl.BlockSpec((bn, 128), lambda i: (i, 0)),
                  pl.BlockSpec((bn, 128), lambda i: (i + nb, 0)),
                  pl.BlockSpec((bn, 128), lambda i: (i, 0)),
                  pl.BlockSpec((bn,), lambda i: (i,)),
                  pl.BlockSpec((128,), lambda i: (0,))],
        out_specs=pl.BlockSpec((bn, 128), lambda i: (i, 0)),
        out_shape=jax.ShapeDtypeStruct((npad, 128), jnp.float32),
    )(acc2, acc2, y2, dinv, b2)


def kernel(x, edge_index, W1, b1, W2, b2):
    n, d = x.shape
    e = edge_index.shape[1]
    npad = -(-(n + 1) // 2048) * 2048
    cpw = -(-e // (NW * EB))
    cpw = -(-cpw // IG) * IG
    ep = NW * cpw * EB

    src = edge_index[0]
    dst = edge_index[1]
    pad = ep - e
    src2d = jnp.concatenate(
        [src, jnp.zeros((pad,), jnp.int32)]).reshape(NW * cpw, EB)
    dst2d = jnp.concatenate(
        [dst, jnp.full((pad,), n, jnp.int32)]).reshape(NW * cpw, EB)
    xp = jnp.pad(x, ((0, npad - n), (0, 0)))

    deg2 = _sc_degree(dst2d, npad, cpw)
    xw1 = _tc_matmul(xp, W1, npad)
    y1, dinv = _tc_scale(deg2, xw1, npad)
    acc1 = _sc_aggregate(y1, src2d, dst2d, npad, cpw)
    y2 = _tc_mid(acc1, y1, dinv, b1, W2, npad)
    acc2 = _sc_aggregate(y2, src2d, dst2d, npad, cpw)
    outp = _tc_final(acc2, y2, dinv, b2, npad)
    return outp[:n]
